# baseline (speedup 1.0000x reference)
import functools

import jax
import jax.numpy as jnp
from jax import lax
from jax.experimental import pallas as pl
from jax.experimental.pallas import tpu as pltpu
from jax.experimental.pallas import tpu_sc as plsc

F = 26
D = 16
B = 4096
V = 4000
ROW = F * D + 32
NW = 32
SPW = B // NW
CH = 4
NCHUNK = SPW // CH
IPC = CH * F


def _ffm_body(w_hbm, x_hbm, out_hbm, idx_v, rows0, rows1, out_v, sem0, sem1):
  wid = lax.axis_index("s") * 2 + lax.axis_index("c")
  iota = jnp.arange(D, dtype=jnp.int32)
  pltpu.sync_copy(x_hbm.at[pl.ds(wid * NCHUNK, NCHUNK)], idx_v)

  onehot = [(iota == k).astype(jnp.float32) for k in range(D)]
  bufs = ((rows0, sem0), (rows1, sem1))

  pltpu.async_copy(w_hbm.at[idx_v.at[0]], rows0, sem0)

  def pair_chunk(c, rows, outvec):
    zero = jnp.zeros((D,), jnp.float32)
    vaccs = (zero, zero, zero, zero)
    for i in range(F - 1):
      ci = i * D

      def jbody(j, acc, i=i, ci=ci):
        cj = j * D
        return tuple(
            acc[s]
            + rows[s * F + i, pl.ds(cj, D)] * rows[s * F + j, pl.ds(ci, D)]
            for s in range(CH)
        )

      vaccs = lax.fori_loop(i + 1, F, jbody, vaccs, unroll=2)
    for s in range(CH):
      lacc = vaccs[s]
      for k in range(16):
        lacc = lacc + rows[s * F + k, pl.ds(F * D, D)] * onehot[k]
      for k in range(16, F):
        lacc = lacc + rows[s * F + k, pl.ds(F * D + 16, D)] * onehot[k - 16]
      z = jnp.sum(lacc)
      outvec = jnp.where(iota == (c % CH) * CH + s, z, outvec)
    return outvec

  def group_body(g, outvec):
    for b in range(2):
      c = g * 2 + b
      rows, sem = bufs[b]
      nrows, nsem = bufs[1 - b]
      pltpu.make_async_copy(w_hbm.at[idx_v.at[c]], rows, sem).wait()

      @pl.when(c + 1 < NCHUNK)
      def _prefetch():
        pltpu.async_copy(w_hbm.at[idx_v.at[c + 1]], nrows, nsem)

      outvec = pair_chunk(c, rows, outvec)

      @pl.when(c % CH == CH - 1)
      def _flush():
        out_v[pl.ds((c // CH) * 16, 16)] = outvec
    return outvec

  lax.fori_loop(0, NCHUNK // 2, group_body, jnp.zeros((D,), jnp.float32))

  for t in range(SPW // D):
    z = out_v[pl.ds(t * D, D)]
    out_v[pl.ds(t * D, D)] = 1.0 / (1.0 + jnp.exp(-z))
  pltpu.sync_copy(out_v, out_hbm.at[pl.ds(wid * SPW, SPW)])


@jax.jit
def _ffm(w_table, x2d):
  mesh = plsc.VectorSubcoreMesh(core_axis_name="c", subcore_axis_name="s")
  return pl.kernel(
      _ffm_body,
      mesh=mesh,
      compiler_params=pltpu.CompilerParams(
          use_tc_tiling_on_sc=False, needs_layout_passes=False
      ),
      out_type=jax.ShapeDtypeStruct((B,), jnp.float32),
      scratch_types=[
          pltpu.VMEM((NCHUNK, IPC), jnp.int32),
          pltpu.VMEM((IPC, ROW), jnp.float32),
          pltpu.VMEM((IPC, ROW), jnp.float32),
          pltpu.VMEM((SPW,), jnp.float32),
          pltpu.SemaphoreType.DMA,
          pltpu.SemaphoreType.DMA,
      ],
  )(w_table, x2d)


def kernel(x, linear_w, bias, emb):
  xi = x.astype(jnp.int32).reshape(B * F // IPC, IPC)
  emb_t = jnp.transpose(emb[:, :V, :], (1, 0, 2)).reshape(V, F * D)
  lin_t = jnp.transpose(linear_w[: F * V, 0].reshape(F, V), (1, 0))
  lin_t = lin_t + bias[0] / F
  w_table = jnp.concatenate(
      [emb_t, lin_t, jnp.zeros((V, ROW - F * D - F), jnp.float32)], axis=1
  )
  return _ffm(w_table, xi)


# speedup vs baseline: 1.0901x; 1.0901x over previous
import functools

import jax
import jax.numpy as jnp
from jax import lax
from jax.experimental import pallas as pl
from jax.experimental.pallas import tpu as pltpu
from jax.experimental.pallas import tpu_sc as plsc

F = 26
D = 16
B = 4096
V = 4000
ROW = F * D + 32
NW = 32
SPW = B // NW
CH = 4
NCHUNK = SPW // CH
IPC = CH * F


def _ffm_body(w_hbm, x_hbm, out_hbm, idx_v, rows0, rows1, out_v, sem0, sem1):
  wid = lax.axis_index("s") * 2 + lax.axis_index("c")
  iota = jnp.arange(D, dtype=jnp.int32)
  pltpu.sync_copy(x_hbm.at[pl.ds(wid * NCHUNK, NCHUNK)], idx_v)

  onehot = [(iota == k).astype(jnp.float32) for k in range(D)]
  bufs = ((rows0, sem0), (rows1, sem1))

  pltpu.async_copy(w_hbm.at[idx_v.at[0]], rows0, sem0)

  def pair_chunk(c, rows, outvec):
    zero = jnp.zeros((D,), jnp.float32)
    vaccs = (zero, zero, zero, zero)
    for i in range(F - 1):
      ci = i * D

      def jbody(j, acc, i=i, ci=ci):
        cj = j * D
        return tuple(
            acc[s]
            + rows[s * F + i, pl.ds(cj, D)] * rows[s * F + j, pl.ds(ci, D)]
            for s in range(CH)
        )

      vaccs = plsc.parallel_loop(i + 1, F, carry=vaccs)(jbody)
    for s in range(CH):
      lacc = vaccs[s]
      for k in range(16):
        lacc = lacc + rows[s * F + k, pl.ds(F * D, D)] * onehot[k]
      for k in range(16, F):
        lacc = lacc + rows[s * F + k, pl.ds(F * D + 16, D)] * onehot[k - 16]
      z = jnp.sum(lacc)
      outvec = jnp.where(iota == (c % CH) * CH + s, z, outvec)
    return outvec

  def group_body(g, outvec):
    for b in range(2):
      c = g * 2 + b
      rows, sem = bufs[b]
      nrows, nsem = bufs[1 - b]
      pltpu.make_async_copy(w_hbm.at[idx_v.at[c]], rows, sem).wait()

      @pl.when(c + 1 < NCHUNK)
      def _prefetch():
        pltpu.async_copy(w_hbm.at[idx_v.at[c + 1]], nrows, nsem)

      outvec = pair_chunk(c, rows, outvec)

      @pl.when(c % CH == CH - 1)
      def _flush():
        out_v[pl.ds((c // CH) * 16, 16)] = outvec
    return outvec

  lax.fori_loop(0, NCHUNK // 2, group_body, jnp.zeros((D,), jnp.float32))

  for t in range(SPW // D):
    z = out_v[pl.ds(t * D, D)]
    out_v[pl.ds(t * D, D)] = 1.0 / (1.0 + jnp.exp(-z))
  pltpu.sync_copy(out_v, out_hbm.at[pl.ds(wid * SPW, SPW)])


@jax.jit
def _ffm(w_table, x2d):
  mesh = plsc.VectorSubcoreMesh(core_axis_name="c", subcore_axis_name="s")
  return pl.kernel(
      _ffm_body,
      mesh=mesh,
      compiler_params=pltpu.CompilerParams(
          use_tc_tiling_on_sc=False, needs_layout_passes=False
      ),
      out_type=jax.ShapeDtypeStruct((B,), jnp.float32),
      scratch_types=[
          pltpu.VMEM((NCHUNK, IPC), jnp.int32),
          pltpu.VMEM((IPC, ROW), jnp.float32),
          pltpu.VMEM((IPC, ROW), jnp.float32),
          pltpu.VMEM((SPW,), jnp.float32),
          pltpu.SemaphoreType.DMA,
          pltpu.SemaphoreType.DMA,
      ],
  )(w_table, x2d)


def kernel(x, linear_w, bias, emb):
  xi = x.astype(jnp.int32).reshape(B * F // IPC, IPC)
  emb_t = jnp.transpose(emb[:, :V, :], (1, 0, 2)).reshape(V, F * D)
  lin_t = jnp.transpose(linear_w[: F * V, 0].reshape(F, V), (1, 0))
  lin_t = lin_t + bias[0] / F
  w_table = jnp.concatenate(
      [emb_t, lin_t, jnp.zeros((V, ROW - F * D - F), jnp.float32)], axis=1
  )
  return _ffm(w_table, xi)
